# baseline (speedup 1.0000x reference)
import functools

import jax
import jax.numpy as jnp
from jax import lax
from jax.experimental import pallas as pl
from jax.experimental.pallas import tpu as pltpu
from jax.experimental.pallas import tpu_sc as plsc

NC, NS, LANES = 2, 16, 16
NW = NC * NS
CD = 4
NB = 4
CUNROLL = 8
TC_FRAC_NUM, TC_FRAC_DEN = 1, 2
TC_BB = 64


def _sc_add_call(tok_flat, pos_flat, Bs, P, row0):
    n_rows = Bs // NW
    CH = P // CD
    SCK = n_rows * CD
    G = SCK // NB
    mesh = plsc.VectorSubcoreMesh(core_axis_name="c", subcore_axis_name="s")

    @functools.partial(
        pl.kernel,
        out_type=jax.ShapeDtypeStruct((Bs * P + row0 * P,), jnp.float32),
        mesh=mesh,
        scratch_types=[
            pltpu.VMEM((P,), jnp.float32),
            *[pltpu.VMEM((CH,), jnp.float32) for _ in range(2 * NB)],
            *[pltpu.SemaphoreType.DMA for _ in range(2 * NB)],
        ],
    )
    def sc_add(tok_hbm, pos_hbm, out_hbm, pos_v, *bufs_and_sems):
        ibs = list(bufs_and_sems[0:NB])
        obs = list(bufs_and_sems[NB:2 * NB])
        sis = list(bufs_and_sems[2 * NB:3 * NB])
        sos = list(bufs_and_sems[3 * NB:4 * NB])
        wid = lax.axis_index("s") * NC + lax.axis_index("c")
        base = (row0 + wid * n_rows) * P
        pltpu.sync_copy(pos_hbm, pos_v)

        def start_in(idx, s):
            pltpu.make_async_copy(
                tok_hbm.at[pl.ds(base + idx * CH, CH)], ibs[s], sis[s]
            ).start()

        def wait_in(s):
            pltpu.make_async_copy(
                tok_hbm.at[pl.ds(0, CH)], ibs[s], sis[s]
            ).wait()

        def start_out(idx, s):
            pltpu.make_async_copy(
                obs[s], out_hbm.at[pl.ds(base + idx * CH, CH)], sos[s]
            ).start()

        def wait_out(s):
            pltpu.make_async_copy(
                obs[s], out_hbm.at[pl.ds(0, CH)], sos[s]
            ).wait()

        def compute(s):
            col = (s % CD) * CH
            ib, ob = ibs[s], obs[s]

            def jbody(j, carry):
                o = j * (LANES * CUNROLL)
                for u in range(CUNROLL):
                    oo = o + u * LANES
                    ob[pl.ds(oo, LANES)] = (
                        ib[pl.ds(oo, LANES)] + pos_v[pl.ds(col + oo, LANES)]
                    )
                return carry

            lax.fori_loop(0, CH // (LANES * CUNROLL), jbody, 0)

        for s in range(NB):
            start_in(s, s)
        for s in range(NB):
            wait_in(s)
            compute(s)
            start_out(s, s)
            start_in(s + NB, s)

        def gbody(g, carry):
            for s in range(NB):
                idx = g * NB + s
                wait_in(s)
                wait_out(s)
                compute(s)
                start_out(idx, s)
                start_in(idx + NB, s)
            return carry

        lax.fori_loop(1, G - 1, gbody, 0)

        for s in range(NB):
            idx = (G - 1) * NB + s
            wait_in(s)
            wait_out(s)
            compute(s)
            start_out(idx, s)
        for s in range(NB):
            wait_out(s)

    return sc_add(tok_flat, pos_flat)


def _tc_body(tok_ref, pos_ref, acc_ref, out_ref):
    del acc_ref
    out_ref[...] = tok_ref[...] + pos_ref[...]


def _tc_add_call(tok, pos, sc_out, Bt):
    B, S, D = tok.shape
    return pl.pallas_call(
        _tc_body,
        grid=(Bt // TC_BB,),
        in_specs=[
            pl.BlockSpec((TC_BB, S, D), lambda i: (i, 0, 0)),
            pl.BlockSpec((S, D), lambda i: (0, 0)),
            pl.BlockSpec(memory_space=pl.ANY),
        ],
        out_specs=pl.BlockSpec((TC_BB, S, D), lambda i: (i, 0, 0)),
        out_shape=jax.ShapeDtypeStruct((B, S, D), tok.dtype),
        input_output_aliases={2: 0},
    )(tok, pos, sc_out)


def kernel(encoded_tokens, pos_table):
    B, S, D = encoded_tokens.shape
    P = S * D
    Bt = (B * TC_FRAC_NUM // TC_FRAC_DEN) // NW * NW
    Bs = B - Bt
    out_sc = _sc_add_call(
        encoded_tokens.reshape(B * P), pos_table.reshape(P), Bs, P, Bt
    )
    return _tc_add_call(encoded_tokens, pos_table, out_sc.reshape(B, S, D), Bt)


# speedup vs baseline: 1.0052x; 1.0052x over previous
import functools

import jax
import jax.numpy as jnp
from jax import lax
from jax.experimental import pallas as pl
from jax.experimental.pallas import tpu as pltpu
from jax.experimental.pallas import tpu_sc as plsc

NC, NS, LANES = 2, 16, 16
NW = NC * NS
CD = 4
NB = 4
CUNROLL = 8
TC_FRAC_NUM, TC_FRAC_DEN = 1, 2
TC_BB = 128


def _sc_add_call(tok_flat, pos_flat, Bs, P, row0):
    n_rows = Bs // NW
    CH = P // CD
    SCK = n_rows * CD
    G = SCK // NB
    mesh = plsc.VectorSubcoreMesh(core_axis_name="c", subcore_axis_name="s")

    @functools.partial(
        pl.kernel,
        out_type=jax.ShapeDtypeStruct((Bs * P + row0 * P,), jnp.float32),
        mesh=mesh,
        scratch_types=[
            pltpu.VMEM((P,), jnp.float32),
            *[pltpu.VMEM((CH,), jnp.float32) for _ in range(2 * NB)],
            *[pltpu.SemaphoreType.DMA for _ in range(2 * NB)],
        ],
    )
    def sc_add(tok_hbm, pos_hbm, out_hbm, pos_v, *bufs_and_sems):
        ibs = list(bufs_and_sems[0:NB])
        obs = list(bufs_and_sems[NB:2 * NB])
        sis = list(bufs_and_sems[2 * NB:3 * NB])
        sos = list(bufs_and_sems[3 * NB:4 * NB])
        wid = lax.axis_index("s") * NC + lax.axis_index("c")
        base = (row0 + wid * n_rows) * P
        pltpu.sync_copy(pos_hbm, pos_v)

        def start_in(idx, s):
            pltpu.make_async_copy(
                tok_hbm.at[pl.ds(base + idx * CH, CH)], ibs[s], sis[s]
            ).start()

        def wait_in(s):
            pltpu.make_async_copy(
                tok_hbm.at[pl.ds(0, CH)], ibs[s], sis[s]
            ).wait()

        def start_out(idx, s):
            pltpu.make_async_copy(
                obs[s], out_hbm.at[pl.ds(base + idx * CH, CH)], sos[s]
            ).start()

        def wait_out(s):
            pltpu.make_async_copy(
                obs[s], out_hbm.at[pl.ds(0, CH)], sos[s]
            ).wait()

        def compute(s):
            col = (s % CD) * CH
            ib, ob = ibs[s], obs[s]

            def jbody(j, carry):
                o = j * (LANES * CUNROLL)
                for u in range(CUNROLL):
                    oo = o + u * LANES
                    ob[pl.ds(oo, LANES)] = (
                        ib[pl.ds(oo, LANES)] + pos_v[pl.ds(col + oo, LANES)]
                    )
                return carry

            lax.fori_loop(0, CH // (LANES * CUNROLL), jbody, 0)

        for s in range(NB):
            start_in(s, s)
        for s in range(NB):
            wait_in(s)
            compute(s)
            start_out(s, s)
            start_in(s + NB, s)

        def gbody(g, carry):
            for s in range(NB):
                idx = g * NB + s
                wait_in(s)
                wait_out(s)
                compute(s)
                start_out(idx, s)
                start_in(idx + NB, s)
            return carry

        lax.fori_loop(1, G - 1, gbody, 0)

        for s in range(NB):
            idx = (G - 1) * NB + s
            wait_in(s)
            wait_out(s)
            compute(s)
            start_out(idx, s)
        for s in range(NB):
            wait_out(s)

    return sc_add(tok_flat, pos_flat)


def _tc_body(tok_ref, pos_ref, acc_ref, out_ref):
    del acc_ref
    out_ref[...] = tok_ref[...] + pos_ref[...]


def _tc_add_call(tok, pos, sc_out, Bt):
    B, S, D = tok.shape
    return pl.pallas_call(
        _tc_body,
        grid=(Bt // TC_BB,),
        in_specs=[
            pl.BlockSpec((TC_BB, S, D), lambda i: (i, 0, 0)),
            pl.BlockSpec((S, D), lambda i: (0, 0)),
            pl.BlockSpec(memory_space=pl.ANY),
        ],
        out_specs=pl.BlockSpec((TC_BB, S, D), lambda i: (i, 0, 0)),
        out_shape=jax.ShapeDtypeStruct((B, S, D), tok.dtype),
        input_output_aliases={2: 0},
    )(tok, pos, sc_out)


def kernel(encoded_tokens, pos_table):
    B, S, D = encoded_tokens.shape
    P = S * D
    Bt = (B * TC_FRAC_NUM // TC_FRAC_DEN) // NW * NW
    Bs = B - Bt
    out_sc = _sc_add_call(
        encoded_tokens.reshape(B * P), pos_table.reshape(P), Bs, P, Bt
    )
    return _tc_add_call(encoded_tokens, pos_table, out_sc.reshape(B, S, D), Bt)


# speedup vs baseline: 1.0056x; 1.0004x over previous
import functools

import jax
import jax.numpy as jnp
from jax import lax
from jax.experimental import pallas as pl
from jax.experimental.pallas import tpu as pltpu
from jax.experimental.pallas import tpu_sc as plsc

NC, NS, LANES = 2, 16, 16
NW = NC * NS
CD = 4
NB = 4
CUNROLL = 8
TC_FRAC_NUM, TC_FRAC_DEN = 1, 2
TC_BB = 128


def _sc_add_call(tok_flat, pos_flat, Bs, P, row0):
    n_rows = Bs // NW
    CH = P // CD
    SCK = n_rows * CD
    G = SCK // NB
    mesh = plsc.VectorSubcoreMesh(core_axis_name="c", subcore_axis_name="s")

    @functools.partial(
        pl.kernel,
        out_type=jax.ShapeDtypeStruct(((row0 + Bs) * P,), jnp.float32),
        mesh=mesh,
        scratch_types=[
            pltpu.VMEM((P,), jnp.float32),
            *[pltpu.VMEM((CH,), jnp.float32) for _ in range(2 * NB)],
            *[pltpu.SemaphoreType.DMA for _ in range(2 * NB)],
        ],
    )
    def sc_add(tok_hbm, pos_hbm, out_hbm, pos_v, *bufs_and_sems):
        ibs = list(bufs_and_sems[0:NB])
        obs = list(bufs_and_sems[NB:2 * NB])
        sis = list(bufs_and_sems[2 * NB:3 * NB])
        sos = list(bufs_and_sems[3 * NB:4 * NB])
        wid = lax.axis_index("s") * NC + lax.axis_index("c")
        base = (row0 + wid * n_rows) * P
        pltpu.sync_copy(pos_hbm, pos_v)

        def start_in(idx, s):
            pltpu.make_async_copy(
                tok_hbm.at[pl.ds(base + idx * CH, CH)], ibs[s], sis[s]
            ).start()

        def wait_in(s):
            pltpu.make_async_copy(
                tok_hbm.at[pl.ds(0, CH)], ibs[s], sis[s]
            ).wait()

        def start_out(idx, s):
            pltpu.make_async_copy(
                obs[s], out_hbm.at[pl.ds(base + idx * CH, CH)], sos[s]
            ).start()

        def wait_out(s):
            pltpu.make_async_copy(
                obs[s], out_hbm.at[pl.ds(0, CH)], sos[s]
            ).wait()

        def compute(s):
            col = (s % CD) * CH
            ib, ob = ibs[s], obs[s]

            def jbody(j, carry):
                o = j * (LANES * CUNROLL)
                for u in range(CUNROLL):
                    oo = o + u * LANES
                    ob[pl.ds(oo, LANES)] = (
                        ib[pl.ds(oo, LANES)] + pos_v[pl.ds(col + oo, LANES)]
                    )
                return carry

            lax.fori_loop(0, CH // (LANES * CUNROLL), jbody, 0)

        for s in range(NB):
            start_in(s, s)
        for s in range(NB):
            wait_in(s)
            compute(s)
            start_out(s, s)
            start_in(s + NB, s)

        def gbody(g, carry):
            for s in range(NB):
                idx = g * NB + s
                wait_in(s)
                wait_out(s)
                compute(s)
                start_out(idx, s)
                start_in(idx + NB, s)
            return carry

        lax.fori_loop(1, G - 1, gbody, 0)

        for s in range(NB):
            idx = (G - 1) * NB + s
            wait_in(s)
            wait_out(s)
            compute(s)
            start_out(idx, s)
        for s in range(NB):
            wait_out(s)

    return sc_add(tok_flat, pos_flat)


def _tc_body(tok_ref, pos_ref, acc_ref, out_ref):
    del acc_ref
    out_ref[...] = tok_ref[...] + pos_ref[...]


def _tc_add_call(tok, pos, sc_out, Bt):
    B, S, D = tok.shape
    return pl.pallas_call(
        _tc_body,
        grid=(Bt // TC_BB,),
        in_specs=[
            pl.BlockSpec((TC_BB, S, D), lambda i: (i, 0, 0)),
            pl.BlockSpec((S, D), lambda i: (0, 0)),
            pl.BlockSpec(memory_space=pl.ANY),
        ],
        out_specs=pl.BlockSpec((TC_BB, S, D), lambda i: (i, 0, 0)),
        out_shape=jax.ShapeDtypeStruct((B, S, D), tok.dtype),
        input_output_aliases={2: 0},
    )(tok, pos, sc_out)


def kernel(encoded_tokens, pos_table):
    B, S, D = encoded_tokens.shape
    P = S * D
    Bt = (B * TC_FRAC_NUM // TC_FRAC_DEN) // NW * NW
    Bs = B - Bt
    out_sc = _sc_add_call(
        encoded_tokens.reshape(B * P), pos_table.reshape(P), Bs, P, Bt
    )
    return _tc_add_call(encoded_tokens, pos_table, out_sc.reshape(B, S, D), Bt)


# speedup vs baseline: 1.0064x; 1.0008x over previous
import functools

import jax
import jax.numpy as jnp
from jax import lax
from jax.experimental import pallas as pl
from jax.experimental.pallas import tpu as pltpu
from jax.experimental.pallas import tpu_sc as plsc

NC, NS, LANES = 2, 16, 16
NW = NC * NS
CD = 4
NB = 4
CUNROLL = 8
TC_FRAC_NUM, TC_FRAC_DEN = 1, 2
TC_BB = 128


def _sc_add_call(tok_flat, pos_flat, Bs, P, row0):
    n_rows = Bs // NW
    CH = P // CD
    SCK = n_rows * CD
    G = SCK // NB
    mesh = plsc.VectorSubcoreMesh(core_axis_name="c", subcore_axis_name="s")

    @functools.partial(
        pl.kernel,
        out_type=jax.ShapeDtypeStruct(((row0 + Bs) * P,), jnp.float32),
        mesh=mesh,
        scratch_types=[
            pltpu.VMEM((P,), jnp.float32),
            *[pltpu.VMEM((CH,), jnp.float32) for _ in range(2 * NB)],
            *[pltpu.SemaphoreType.DMA for _ in range(2 * NB)],
        ],
    )
    def sc_add(tok_hbm, pos_hbm, out_hbm, pos_v, *bufs_and_sems):
        ibs = list(bufs_and_sems[0:NB])
        obs = list(bufs_and_sems[NB:2 * NB])
        sis = list(bufs_and_sems[2 * NB:3 * NB])
        sos = list(bufs_and_sems[3 * NB:4 * NB])
        wid = lax.axis_index("s") * NC + lax.axis_index("c")
        base = (row0 + wid * n_rows) * P
        pltpu.sync_copy(pos_hbm, pos_v)

        def start_in(idx, s):
            pltpu.make_async_copy(
                tok_hbm.at[pl.ds(base + idx * CH, CH)], ibs[s], sis[s]
            ).start()

        def wait_in(s):
            pltpu.make_async_copy(
                tok_hbm.at[pl.ds(0, CH)], ibs[s], sis[s]
            ).wait()

        def start_out(idx, s):
            pltpu.make_async_copy(
                obs[s], out_hbm.at[pl.ds(base + idx * CH, CH)], sos[s]
            ).start()

        def wait_out(s):
            pltpu.make_async_copy(
                obs[s], out_hbm.at[pl.ds(0, CH)], sos[s]
            ).wait()

        def compute(s):
            col = (s % CD) * CH
            ib, ob = ibs[s], obs[s]

            def jbody(j, carry):
                o = j * (LANES * CUNROLL)
                for u in range(CUNROLL):
                    oo = o + u * LANES
                    ob[pl.ds(oo, LANES)] = (
                        ib[pl.ds(oo, LANES)] + pos_v[pl.ds(col + oo, LANES)]
                    )
                return carry

            lax.fori_loop(0, CH // (LANES * CUNROLL), jbody, 0)

        for s in range(NB):
            start_in(s, s)
        for s in range(NB):
            wait_in(s)
            compute(s)
            start_out(s, s)
            start_in(s + NB, s)

        def gbody(g, carry):
            for s in range(NB):
                idx = g * NB + s
                wait_in(s)
                wait_out(s)
                compute(s)
                start_out(idx, s)
                start_in(idx + NB, s)
            return carry

        lax.fori_loop(1, G - 1, gbody, 0)

        for s in range(NB):
            idx = (G - 1) * NB + s
            wait_in(s)
            wait_out(s)
            compute(s)
            start_out(idx, s)
        for s in range(NB):
            wait_out(s)

    return sc_add(tok_flat, pos_flat)


def _tc_body(tok_ref, pos_ref, acc_ref, out_ref):
    del acc_ref
    out_ref[...] = tok_ref[...] + pos_ref[...]


def _tc_add_call(tok, pos, sc_out, Bt):
    B, S, D = tok.shape
    return pl.pallas_call(
        _tc_body,
        grid=(Bt // TC_BB,),
        in_specs=[
            pl.BlockSpec((TC_BB, S, D), lambda i: (i, 0, 0)),
            pl.BlockSpec((S, D), lambda i: (0, 0)),
            pl.BlockSpec(memory_space=pl.ANY),
        ],
        out_specs=pl.BlockSpec((TC_BB, S, D), lambda i: (i, 0, 0)),
        out_shape=jax.ShapeDtypeStruct((B, S, D), tok.dtype),
        input_output_aliases={2: 0},
        compiler_params=pltpu.CompilerParams(
            dimension_semantics=("arbitrary",)),
    )(tok, pos, sc_out)


def kernel(encoded_tokens, pos_table):
    B, S, D = encoded_tokens.shape
    P = S * D
    Bt = (B * TC_FRAC_NUM // TC_FRAC_DEN) // NW * NW
    Bs = B - Bt
    out_sc = _sc_add_call(
        encoded_tokens.reshape(B * P), pos_table.reshape(P), Bs, P, Bt
    )
    return _tc_add_call(encoded_tokens, pos_table, out_sc.reshape(B, S, D), Bt)
